# baseline (speedup 1.0000x reference)
import functools

import jax
import jax.numpy as jnp
from jax import lax
from jax.experimental import pallas as pl
from jax.experimental.pallas import tpu as pltpu, tpu_sc as plsc

M = 1025
NLAGS = 63
PADLEN = 2048
NCHUNK = 65
NTAIL = 63
PAD = 480


def _gather16(x, idx):
    dnums = lax.GatherDimensionNumbers(
        offset_dims=(), collapsed_slice_dims=(0,), start_index_map=(0,))
    return lax.gather(x, idx[:, None], dnums, (1,),
                      mode=lax.GatherScatterMode.PROMISE_IN_BOUNDS)


def _lanesum(x):
    lane = lax.iota(jnp.int32, 16)
    for sh in (8, 4, 2, 1):
        x = x + _gather16(x, jnp.bitwise_xor(lane, sh))
    return x


def _sc_corr(wtx_hbm, wrx_hbm, out_hbm, stg_hbm,
             wtx_v, wrx_v, za_v, ca_v, cb_v, stage_v):
    c = lax.axis_index("c")
    s = lax.axis_index("s")

    pltpu.sync_copy(wtx_hbm, wtx_v.at[pl.ds(0, M)])
    pltpu.sync_copy(wrx_hbm, wrx_v.at[pl.ds(0, M)])

    zero = jnp.zeros((16,), jnp.float32)

    def ztail(i, _):
        wtx_v[pl.ds(M + 16 * i, 16)] = zero
        wrx_v[pl.ds(M + 16 * i, 16)] = zero
        return 0

    lax.fori_loop(0, NTAIL, ztail, 0)

    j_a = 16 * c + s
    j_b = j_a + 32
    r0 = 32 * (31 - j_a)
    t1 = 32 * (jnp.minimum(j_b, NLAGS - 1) - 31)

    def body(i, carry):
        at, ar, aa, ab = carry
        b = i * 16
        t = wtx_v[pl.ds(b, 16)]
        r = wrx_v[pl.ds(b, 16)]
        at = at + t * t
        ar = ar + r * r
        aa = aa + t * wrx_v[pl.ds(b + r0, 16)]
        ab = ab + r * wtx_v[pl.ds(b + t1, 16)]
        return at, ar, aa, ab

    at, ar, aa, ab = lax.fori_loop(0, NCHUNK, body, (zero, zero, zero, zero))

    st = _lanesum(at)
    sr = _lanesum(ar)
    sa = _lanesum(aa)
    sb = _lanesum(ab) * jnp.where(j_b <= NLAGS - 1, 1.0, 0.0)

    p = (st * sr)[0]
    iv = lax.bitcast_convert_type(p, jnp.int32)
    y = lax.bitcast_convert_type(
        jnp.int32(0x5F3759DF) - lax.shift_right_logical(iv, 1), jnp.float32)
    half_p = 0.5 * p
    for _ in range(3):
        y = y * (1.5 - half_p * y * y)

    lane = lax.iota(jnp.int32, 16)
    one = jnp.ones((16,), jnp.float32)

    hot = jnp.where(lane == s, one, zero)
    ca_v[...] = sa * y * hot
    cb_v[...] = sb * y * hot
    pltpu.sync_copy(ca_v, stg_hbm.at[32 * c + s])
    pltpu.sync_copy(cb_v, stg_hbm.at[32 * c + 16 + s])

    za_v[...] = zero
    rid = j_a
    not_val = jnp.logical_and(rid != 15, rid != 16)

    @pl.when(not_val)
    def _():
        pltpu.sync_copy(za_v, out_hbm.at[2 * rid])
        pltpu.sync_copy(za_v, out_hbm.at[2 * rid + 1])

    plsc.subcore_barrier()

    @pl.when(s <= 1)
    def _():
        pltpu.sync_copy(stg_hbm.at[pl.ds(32 * c + 16 * s, 16)], stage_v)
        acc = stage_v[0, :]
        for k in range(1, 16):
            acc = acc + stage_v[k, :]
        ca_v[...] = acc
        pltpu.sync_copy(ca_v, out_hbm.at[30 + c + 2 * s])


@jax.jit
def _run(w_tx, w_rx):
    mesh = plsc.VectorSubcoreMesh(core_axis_name="c", subcore_axis_name="s")
    f = functools.partial(
        pl.kernel,
        out_type=(
            jax.ShapeDtypeStruct((64, 16), jnp.float32),
            jax.ShapeDtypeStruct((64, 16), jnp.float32),
        ),
        mesh=mesh,
        scratch_types=[
            pltpu.VMEM((PADLEN,), jnp.float32),
            pltpu.VMEM((PADLEN,), jnp.float32),
            pltpu.VMEM((16,), jnp.float32),
            pltpu.VMEM((16,), jnp.float32),
            pltpu.VMEM((16,), jnp.float32),
            pltpu.VMEM((16, 16), jnp.float32),
        ],
    )(_sc_corr)
    return f(w_tx, w_rx)[0]


def kernel(W_tx, W_rx, L):
    rows = _run(W_tx, W_rx)
    a = rows.reshape(1024)[: 2 * PAD + NLAGS]
    return lax.complex(a, jnp.zeros_like(a))


# speedup vs baseline: 1.1549x; 1.1549x over previous
import functools

import jax
import jax.numpy as jnp
from jax import lax
from jax.experimental import pallas as pl
from jax.experimental.pallas import tpu as pltpu, tpu_sc as plsc

M = 1025
NLAGS = 63
PADLEN = 2048
NCHUNK = 65
PAD = 480


def _gather16(x, idx):
    dnums = lax.GatherDimensionNumbers(
        offset_dims=(), collapsed_slice_dims=(0,), start_index_map=(0,))
    return lax.gather(x, idx[:, None], dnums, (1,),
                      mode=lax.GatherScatterMode.PROMISE_IN_BOUNDS)


def _lanesum(x, lane):
    for sh in (8, 4, 2, 1):
        x = x + _gather16(x, jnp.bitwise_xor(lane, sh))
    return x


def _sc_corr(wtx_hbm, wrx_hbm, out_hbm, wtx_v, wrx_v, res_v, sem1, sem2):
    s = lax.axis_index("s")

    cp1 = pltpu.async_copy(wtx_hbm, wtx_v, sem1)
    cp2 = pltpu.async_copy(wrx_hbm, wrx_v, sem2)
    cp1.wait()
    cp2.wait()

    r0 = 32 * (31 - s)
    r1 = 32 * (15 - s)
    t2 = 32 * (s + 1)
    t3 = 32 * (jnp.minimum(s + 17, 31))

    zero = jnp.zeros((16,), jnp.float32)

    def body(i, carry):
        at, ar, a0, a1, a2, a3 = carry
        b = i * 16
        t = wtx_v[pl.ds(b, 16)]
        r = wrx_v[pl.ds(b, 16)]
        at = at + t * t
        ar = ar + r * r
        a0 = a0 + t * wrx_v[pl.ds(b + r0, 16)]
        a1 = a1 + t * wrx_v[pl.ds(b + r1, 16)]
        a2 = a2 + r * wtx_v[pl.ds(b + t2, 16)]
        a3 = a3 + r * wtx_v[pl.ds(b + t3, 16)]
        return at, ar, a0, a1, a2, a3

    at, ar, a0, a1, a2, a3 = lax.fori_loop(
        0, NCHUNK, body, (zero,) * 6)

    lane = lax.iota(jnp.int32, 16)
    st = _lanesum(at, lane)
    sr = _lanesum(ar, lane)
    s0 = _lanesum(a0, lane)
    s1 = _lanesum(a1, lane)
    s2 = _lanesum(a2, lane)
    s3 = _lanesum(a3, lane) * jnp.where(s + 48 <= NLAGS - 1, 1.0, 0.0)

    p = (st * sr)[0]
    iv = lax.bitcast_convert_type(p, jnp.int32)
    y = lax.bitcast_convert_type(
        jnp.int32(0x5F3759DF) - lax.shift_right_logical(iv, 1), jnp.float32)
    half_p = 0.5 * p
    for _ in range(3):
        y = y * (1.5 - half_p * y * y)

    one = jnp.ones((16,), jnp.float32)
    res = s0 * jnp.where(lane == 0, one, zero)
    res = res + s1 * jnp.where(lane == 1, one, zero)
    res = res + s2 * jnp.where(lane == 2, one, zero)
    res = res + s3 * jnp.where(lane == 3, one, zero)
    res_v[...] = res * y
    pltpu.sync_copy(res_v, out_hbm.at[s])


@jax.jit
def _run(wtx_pad, wrx_pad):
    mesh = plsc.VectorSubcoreMesh(
        core_axis_name="c", subcore_axis_name="s", num_cores=1)
    f = functools.partial(
        pl.kernel,
        out_type=jax.ShapeDtypeStruct((16, 16), jnp.float32),
        mesh=mesh,
        scratch_types=[
            pltpu.VMEM((PADLEN,), jnp.float32),
            pltpu.VMEM((PADLEN,), jnp.float32),
            pltpu.VMEM((16,), jnp.float32),
            pltpu.SemaphoreType.DMA,
            pltpu.SemaphoreType.DMA,
        ],
    )(_sc_corr)
    return f(wtx_pad, wrx_pad)


def kernel(W_tx, W_rx, L):
    wtx_pad = jnp.zeros((PADLEN,), jnp.float32).at[:M].set(W_tx)
    wrx_pad = jnp.zeros((PADLEN,), jnp.float32).at[:M].set(W_rx)
    rows = _run(wtx_pad, wrx_pad)
    vals = rows[:, :4].T.reshape(64)[:NLAGS]
    z = jnp.zeros((PAD,), jnp.float32)
    a = jnp.concatenate([z, vals, z])
    return lax.complex(a, jnp.zeros_like(a))
